# baseline (speedup 1.0000x reference)
import functools

import jax
import jax.numpy as jnp
from jax.experimental import pallas as pl
from jax.experimental.pallas import tpu as pltpu

_EMB = 512
_M = 4096


def _gather_body(idx_ref, w_ref, out_ref, buf0, buf1, gsem, wsem, *, nsteps):
    k = pl.program_id(0)
    bufs = (buf0, buf1)

    for p in (0, 1):
        buf = bufs[p]

        @pl.when(jnp.logical_and(k < nsteps, k % 2 == p))
        def _issue():
            @pl.when(k >= 2)
            def _wait_write():
                pltpu.make_async_copy(buf, out_ref.at[pl.ds(0, _M)], wsem.at[p]).wait()

            base = k * _M
            for m in range(_M):
                row = idx_ref[base + m]
                pltpu.make_async_copy(
                    w_ref.at[pl.ds(row, 1)],
                    buf.at[pl.ds(m, 1)],
                    gsem.at[2 * p + (m % 2)],
                ).start()

        @pl.when(jnp.logical_and(k >= 1, k % 2 == p))
        def _flush_prev():
            prev = bufs[1 - p]
            half = w_ref.at[pl.ds(0, _M // 2)], prev.at[pl.ds(0, _M // 2)]
            pltpu.make_async_copy(*half, gsem.at[2 * (1 - p)]).wait()
            pltpu.make_async_copy(*half, gsem.at[2 * (1 - p) + 1]).wait()
            pltpu.make_async_copy(
                prev, out_ref.at[pl.ds((k - 1) * _M, _M)], wsem.at[1 - p]
            ).start()

    @pl.when(k == nsteps)
    def _final():
        pltpu.make_async_copy(buf0, out_ref.at[pl.ds(0, _M)], wsem.at[0]).wait()
        pltpu.make_async_copy(buf1, out_ref.at[pl.ds(0, _M)], wsem.at[1]).wait()


def kernel(input, weight):
    bptt, bsize = input.shape
    n = bptt * bsize
    idx = input.reshape(n).astype(jnp.int32)
    nsteps = n // _M

    grid_spec = pltpu.PrefetchScalarGridSpec(
        num_scalar_prefetch=1,
        grid=(nsteps + 1,),
        in_specs=[pl.BlockSpec(memory_space=pl.ANY)],
        out_specs=pl.BlockSpec(memory_space=pl.ANY),
        scratch_shapes=[
            pltpu.VMEM((_M, _EMB), jnp.float32),
            pltpu.VMEM((_M, _EMB), jnp.float32),
            pltpu.SemaphoreType.DMA((4,)),
            pltpu.SemaphoreType.DMA((2,)),
        ],
    )
    out = pl.pallas_call(
        functools.partial(_gather_body, nsteps=nsteps),
        grid_spec=grid_spec,
        out_shape=jax.ShapeDtypeStruct((n, _EMB), jnp.float32),
        compiler_params=pltpu.CompilerParams(
            dimension_semantics=("arbitrary",),
            disable_bounds_checks=True,
        ),
    )(idx, weight)
    return out.reshape(bptt, bsize, _EMB)


# speedup vs baseline: 1.0227x; 1.0227x over previous
import functools

import jax
import jax.numpy as jnp
from jax.experimental import pallas as pl
from jax.experimental.pallas import tpu as pltpu

_EMB = 512
_M = 2048


def _gather_body(idx_ref, w_ref, out_ref, buf0, buf1, gsem, wsem, *, nsteps):
    k = pl.program_id(0)
    bufs = (buf0, buf1)

    for p in (0, 1):
        buf = bufs[p]

        @pl.when(jnp.logical_and(k < nsteps, k % 2 == p))
        def _issue():
            @pl.when(k >= 2)
            def _wait_write():
                pltpu.make_async_copy(buf, out_ref.at[pl.ds(0, _M)], wsem.at[p]).wait()

            base = k * _M
            for m in range(_M):
                row = idx_ref[base + m]
                pltpu.make_async_copy(
                    w_ref.at[pl.ds(row, 1)],
                    buf.at[pl.ds(m, 1)],
                    gsem.at[2 * p + (m % 2)],
                ).start()

        @pl.when(jnp.logical_and(k >= 1, k % 2 == p))
        def _flush_prev():
            prev = bufs[1 - p]
            half = w_ref.at[pl.ds(0, _M // 2)], prev.at[pl.ds(0, _M // 2)]
            pltpu.make_async_copy(*half, gsem.at[2 * (1 - p)]).wait()
            pltpu.make_async_copy(*half, gsem.at[2 * (1 - p) + 1]).wait()
            pltpu.make_async_copy(
                prev, out_ref.at[pl.ds((k - 1) * _M, _M)], wsem.at[1 - p]
            ).start(priority=1)

    @pl.when(k == nsteps)
    def _final():
        pltpu.make_async_copy(buf0, out_ref.at[pl.ds(0, _M)], wsem.at[0]).wait()
        pltpu.make_async_copy(buf1, out_ref.at[pl.ds(0, _M)], wsem.at[1]).wait()


def kernel(input, weight):
    bptt, bsize = input.shape
    n = bptt * bsize
    idx = input.reshape(n).astype(jnp.int32)
    nsteps = n // _M

    grid_spec = pltpu.PrefetchScalarGridSpec(
        num_scalar_prefetch=1,
        grid=(nsteps + 1,),
        in_specs=[pl.BlockSpec(memory_space=pl.ANY)],
        out_specs=pl.BlockSpec(memory_space=pl.ANY),
        scratch_shapes=[
            pltpu.VMEM((_M, _EMB), jnp.float32),
            pltpu.VMEM((_M, _EMB), jnp.float32),
            pltpu.SemaphoreType.DMA((4,)),
            pltpu.SemaphoreType.DMA((2,)),
        ],
    )
    out = pl.pallas_call(
        functools.partial(_gather_body, nsteps=nsteps),
        grid_spec=grid_spec,
        out_shape=jax.ShapeDtypeStruct((n, _EMB), jnp.float32),
        compiler_params=pltpu.CompilerParams(
            dimension_semantics=("arbitrary",),
            disable_bounds_checks=True,
        ),
    )(idx, weight)
    return out.reshape(bptt, bsize, _EMB)


# speedup vs baseline: 1.0252x; 1.0025x over previous
import functools

import jax
import jax.numpy as jnp
from jax.experimental import pallas as pl
from jax.experimental.pallas import tpu as pltpu

_EMB = 512
_M = 2048


def _gather_body(idx_ref, w_ref, out_ref, buf0, buf1, gsem, wsem, *, nsteps):
    k = pl.program_id(0)
    bufs = (buf0, buf1)

    for p in (0, 1):
        buf = bufs[p]

        @pl.when(jnp.logical_and(k < nsteps, k % 2 == p))
        def _issue():
            @pl.when(k >= 2)
            def _wait_write():
                pltpu.make_async_copy(buf, out_ref.at[pl.ds(0, _M)], wsem.at[p]).wait()

            base = k * _M
            for m in range(_M):
                row = idx_ref[base + m]
                pltpu.make_async_copy(
                    w_ref.at[pl.ds(row, 1)],
                    buf.at[pl.ds(m, 1)],
                    gsem.at[4 * p + m // (_M // 4)],
                ).start()

        @pl.when(jnp.logical_and(k >= 1, k % 2 == p))
        def _flush_prev():
            prev = bufs[1 - p]
            q = _M // 4
            for qi in range(4):
                pltpu.make_async_copy(
                    w_ref.at[pl.ds(0, q)], prev.at[pl.ds(0, q)],
                    gsem.at[4 * (1 - p) + qi],
                ).wait()
                pltpu.make_async_copy(
                    prev.at[pl.ds(qi * q, q)],
                    out_ref.at[pl.ds((k - 1) * _M + qi * q, q)],
                    wsem.at[1 - p],
                ).start()

    @pl.when(k == nsteps)
    def _final():
        pltpu.make_async_copy(buf0, out_ref.at[pl.ds(0, _M)], wsem.at[0]).wait()
        pltpu.make_async_copy(buf1, out_ref.at[pl.ds(0, _M)], wsem.at[1]).wait()


def kernel(input, weight):
    bptt, bsize = input.shape
    n = bptt * bsize
    idx = input.reshape(n).astype(jnp.int32)
    nsteps = n // _M

    grid_spec = pltpu.PrefetchScalarGridSpec(
        num_scalar_prefetch=1,
        grid=(nsteps + 1,),
        in_specs=[pl.BlockSpec(memory_space=pl.ANY)],
        out_specs=pl.BlockSpec(memory_space=pl.ANY),
        scratch_shapes=[
            pltpu.VMEM((_M, _EMB), jnp.float32),
            pltpu.VMEM((_M, _EMB), jnp.float32),
            pltpu.SemaphoreType.DMA((8,)),
            pltpu.SemaphoreType.DMA((2,)),
        ],
    )
    out = pl.pallas_call(
        functools.partial(_gather_body, nsteps=nsteps),
        grid_spec=grid_spec,
        out_shape=jax.ShapeDtypeStruct((n, _EMB), jnp.float32),
        compiler_params=pltpu.CompilerParams(
            dimension_semantics=("arbitrary",),
            disable_bounds_checks=True,
        ),
    )(idx, weight)
    return out.reshape(bptt, bsize, _EMB)


# speedup vs baseline: 1.0416x; 1.0159x over previous
import functools

import jax
import jax.numpy as jnp
from jax.experimental import pallas as pl
from jax.experimental.pallas import tpu as pltpu

_EMB = 512
_M = 1024
_NBUF = 3


def _gather_body(idx_ref, w_ref, out_ref, buf0, buf1, buf2, gsem, wsem, *, nsteps):
    k = pl.program_id(0)
    bufs = (buf0, buf1, buf2)

    for p in range(_NBUF):
        buf = bufs[p]

        @pl.when(jnp.logical_and(k < nsteps, k % _NBUF == p))
        def _issue():
            @pl.when(k >= _NBUF)
            def _wait_write():
                pltpu.make_async_copy(buf, out_ref.at[pl.ds(0, _M)], wsem.at[p]).wait()

            base = k * _M
            for m in range(_M):
                row = idx_ref[base + m]
                pltpu.make_async_copy(
                    w_ref.at[pl.ds(row, 1)],
                    buf.at[pl.ds(m, 1)],
                    gsem.at[p],
                ).start()

        @pl.when(jnp.logical_and(k >= 2, (k - 2) % _NBUF == p))
        def _flush_prev():
            pltpu.make_async_copy(w_ref.at[pl.ds(0, _M)], buf, gsem.at[p]).wait()
            pltpu.make_async_copy(
                buf, out_ref.at[pl.ds((k - 2) * _M, _M)], wsem.at[p]
            ).start()

    @pl.when(k == nsteps + 1)
    def _final():
        for p in range(_NBUF):
            pltpu.make_async_copy(bufs[p], out_ref.at[pl.ds(0, _M)], wsem.at[p]).wait()


def kernel(input, weight):
    bptt, bsize = input.shape
    n = bptt * bsize
    idx = input.reshape(n).astype(jnp.int32)
    nsteps = n // _M

    grid_spec = pltpu.PrefetchScalarGridSpec(
        num_scalar_prefetch=1,
        grid=(nsteps + 2,),
        in_specs=[pl.BlockSpec(memory_space=pl.ANY)],
        out_specs=pl.BlockSpec(memory_space=pl.ANY),
        scratch_shapes=[
            pltpu.VMEM((_M, _EMB), jnp.float32),
            pltpu.VMEM((_M, _EMB), jnp.float32),
            pltpu.VMEM((_M, _EMB), jnp.float32),
            pltpu.SemaphoreType.DMA((_NBUF,)),
            pltpu.SemaphoreType.DMA((_NBUF,)),
        ],
    )
    out = pl.pallas_call(
        functools.partial(_gather_body, nsteps=nsteps),
        grid_spec=grid_spec,
        out_shape=jax.ShapeDtypeStruct((n, _EMB), jnp.float32),
        compiler_params=pltpu.CompilerParams(
            dimension_semantics=("arbitrary",),
            disable_bounds_checks=True,
        ),
    )(idx, weight)
    return out.reshape(bptt, bsize, _EMB)


# speedup vs baseline: 1.0446x; 1.0029x over previous
import functools

import jax
import jax.numpy as jnp
from jax.experimental import pallas as pl
from jax.experimental.pallas import tpu as pltpu

_EMB = 512
_M = 2048
_NBUF = 3


def _gather_body(idx_ref, w_ref, out_ref, buf0, buf1, buf2, gsem, wsem, *, nsteps):
    k = pl.program_id(0)
    bufs = (buf0, buf1, buf2)

    for p in range(_NBUF):
        buf = bufs[p]

        @pl.when(jnp.logical_and(k < nsteps, k % _NBUF == p))
        def _issue():
            @pl.when(k >= _NBUF)
            def _wait_write():
                pltpu.make_async_copy(buf, out_ref.at[pl.ds(0, _M)], wsem.at[p]).wait()

            base = k * _M
            for m in range(_M):
                row = idx_ref[base + m]
                pltpu.make_async_copy(
                    w_ref.at[pl.ds(row, 1)],
                    buf.at[pl.ds(m, 1)],
                    gsem.at[p],
                ).start()

        @pl.when(jnp.logical_and(k >= 2, (k - 2) % _NBUF == p))
        def _flush_prev():
            pltpu.make_async_copy(w_ref.at[pl.ds(0, _M)], buf, gsem.at[p]).wait()
            pltpu.make_async_copy(
                buf, out_ref.at[pl.ds((k - 2) * _M, _M)], wsem.at[p]
            ).start()

    @pl.when(k == nsteps + 1)
    def _final():
        for p in range(_NBUF):
            pltpu.make_async_copy(bufs[p], out_ref.at[pl.ds(0, _M)], wsem.at[p]).wait()


def kernel(input, weight):
    bptt, bsize = input.shape
    n = bptt * bsize
    idx = input.reshape(n).astype(jnp.int32)
    nsteps = n // _M

    grid_spec = pltpu.PrefetchScalarGridSpec(
        num_scalar_prefetch=1,
        grid=(nsteps + 2,),
        in_specs=[pl.BlockSpec(memory_space=pl.ANY)],
        out_specs=pl.BlockSpec(memory_space=pl.ANY),
        scratch_shapes=[
            pltpu.VMEM((_M, _EMB), jnp.float32),
            pltpu.VMEM((_M, _EMB), jnp.float32),
            pltpu.VMEM((_M, _EMB), jnp.float32),
            pltpu.SemaphoreType.DMA((_NBUF,)),
            pltpu.SemaphoreType.DMA((_NBUF,)),
        ],
    )
    out = pl.pallas_call(
        functools.partial(_gather_body, nsteps=nsteps),
        grid_spec=grid_spec,
        out_shape=jax.ShapeDtypeStruct((n, _EMB), jnp.float32),
        compiler_params=pltpu.CompilerParams(
            dimension_semantics=("arbitrary",),
            disable_bounds_checks=True,
        ),
    )(idx, weight)
    return out.reshape(bptt, bsize, _EMB)


# speedup vs baseline: 1.0948x; 1.0480x over previous
import functools

import jax
import jax.numpy as jnp
from jax.experimental import pallas as pl
from jax.experimental.pallas import tpu as pltpu

_EMB = 512
_M = 1024
_NBUF = 4


def _gather_body(idx_ref, w_ref, out_ref, buf0, buf1, buf2, buf3, gsem, wsem, *, nsteps):
    k = pl.program_id(0)
    bufs = (buf0, buf1, buf2, buf3)

    for p in range(_NBUF):
        buf = bufs[p]

        @pl.when(jnp.logical_and(k < nsteps, k % _NBUF == p))
        def _issue():
            @pl.when(k >= _NBUF)
            def _wait_write():
                pltpu.make_async_copy(buf, out_ref.at[pl.ds(0, _M)], wsem.at[p]).wait()

            base = k * _M
            for m in range(_M):
                row = idx_ref[base + m]
                pltpu.make_async_copy(
                    w_ref.at[pl.ds(row, 1)],
                    buf.at[pl.ds(m, 1)],
                    gsem.at[p],
                ).start()

        @pl.when(jnp.logical_and(k >= 2, (k - 2) % _NBUF == p))
        def _flush_prev():
            pltpu.make_async_copy(w_ref.at[pl.ds(0, _M)], buf, gsem.at[p]).wait()
            pltpu.make_async_copy(
                buf, out_ref.at[pl.ds((k - 2) * _M, _M)], wsem.at[p]
            ).start()

    @pl.when(k == nsteps + 1)
    def _final():
        for p in range(_NBUF):
            pltpu.make_async_copy(bufs[p], out_ref.at[pl.ds(0, _M)], wsem.at[p]).wait()


def kernel(input, weight):
    bptt, bsize = input.shape
    n = bptt * bsize
    idx = input.reshape(n).astype(jnp.int32)
    nsteps = n // _M

    grid_spec = pltpu.PrefetchScalarGridSpec(
        num_scalar_prefetch=1,
        grid=(nsteps + 2,),
        in_specs=[pl.BlockSpec(memory_space=pl.ANY)],
        out_specs=pl.BlockSpec(memory_space=pl.ANY),
        scratch_shapes=[
            pltpu.VMEM((_M, _EMB), jnp.float32),
            pltpu.VMEM((_M, _EMB), jnp.float32),
            pltpu.VMEM((_M, _EMB), jnp.float32),
            pltpu.VMEM((_M, _EMB), jnp.float32),
            pltpu.SemaphoreType.DMA((_NBUF,)),
            pltpu.SemaphoreType.DMA((_NBUF,)),
        ],
    )
    out = pl.pallas_call(
        functools.partial(_gather_body, nsteps=nsteps),
        grid_spec=grid_spec,
        out_shape=jax.ShapeDtypeStruct((n, _EMB), jnp.float32),
        compiler_params=pltpu.CompilerParams(
            dimension_semantics=("arbitrary",),
            disable_bounds_checks=True,
        ),
    )(idx, weight)
    return out.reshape(bptt, bsize, _EMB)
